# Initial kernel scaffold; baseline (speedup 1.0000x reference)
import functools

import jax
import jax.numpy as jnp
from jax import lax
from jax.experimental import pallas as pl
from jax.experimental.pallas import tpu as pltpu
from jax.experimental.pallas import tpu_sc as plsc

_L = 16
_NS = 16
_NC = 2
_G = 64


def _leaky_relu(v):
    return jnp.where(v >= 0, v, 0.01 * v)


@functools.lru_cache(maxsize=None)
def _sc_aggregate(N, D, E):
    HALF = N // 2
    SLAB = ((HALF + _NS * _L - 1) // (_NS * _L)) * (_NS * _L)
    if SLAB < HALF + 1:
        SLAB += _NS * _L
    RPT = SLAB // _NS
    C = 128
    NCH = E // C
    WB = 40
    NB = HALF // WB

    mesh = plsc.VectorSubcoreMesh(
        core_axis_name="c", subcore_axis_name="s",
        num_cores=_NC, num_subcores=_NS)

    @functools.partial(
        pl.kernel,
        out_type=jax.ShapeDtypeStruct((N, D), jnp.float32),
        mesh=mesh,
        scratch_types=[
            pltpu.VMEM_SHARED((SLAB, D), jnp.float32),
            pltpu.VMEM((C,), jnp.int32),
            pltpu.VMEM((C,), jnp.int32),
            pltpu.VMEM((C, D), jnp.float32),
            pltpu.VMEM((_L, D), jnp.float32),
            pltpu.SemaphoreType.DMA,
        ],
    )
    def agg(p_hbm, src_hbm, dst_hbm, out_hbm, slab, src_v, dst_v, rows_v,
            zero_v, sem):
        c = lax.axis_index("c")
        s = lax.axis_index("s")
        base_row = c * HALF

        def _zv(t, _):
            i = t // (D // _L)
            j = t % (D // _L)
            zero_v[i, pl.ds(j * _L, _L)] = jnp.zeros((_L,), jnp.float32)
            return 0
        lax.fori_loop(0, _L * (D // _L), _zv, 0)

        def _zs(k, _):
            pltpu.sync_copy(zero_v, slab.at[pl.ds(s * RPT + k * _L, _L)])
            return 0
        lax.fori_loop(0, RPT // _L, _zs, 0)
        plsc.subcore_barrier()

        n_my = NCH // _NS + jnp.where(s < (NCH % _NS), 1, 0)

        def _chunk(i, _):
            eb = (i * _NS + s) * C
            pltpu.sync_copy(src_hbm.at[pl.ds(eb, C)], src_v)
            pltpu.sync_copy(dst_hbm.at[pl.ds(eb, C)], dst_v)
            pltpu.async_copy(p_hbm.at[src_v], rows_v, sem).wait()

            def _remap(j, __):
                d = dst_v[pl.ds(j * _L, _L)] - base_row
                oob = jnp.logical_or(d < 0, d >= HALF)
                dst_v[pl.ds(j * _L, _L)] = jnp.where(oob, HALF, d)
                return 0
            lax.fori_loop(0, C // _L, _remap, 0)

            pltpu.sync_copy(rows_v, slab.at[dst_v], add=True)
            return 0
        lax.fori_loop(0, n_my, _chunk, 0)
        plsc.subcore_barrier()

        n_wb = NB // _NS + jnp.where(s < (NB % _NS), 1, 0)

        def _wout(j, _):
            rb = (j * _NS + s) * WB
            pltpu.sync_copy(slab.at[pl.ds(rb, WB)],
                            out_hbm.at[pl.ds(base_row + rb, WB)])
            return 0
        lax.fori_loop(0, n_wb, _wout, 0)

    return agg


_RB = 1000


def _stage1_body(x_ref, ac_ref, w1l_ref, w1r_ref, b1_ref, W2l_ref, W2r_ref,
                 p1_ref, r1_ref):
    a = ac_ref[:, 0:1]
    h1 = a * w1l_ref[...] + x_ref[...] * w1r_ref[...] + b1_ref[...]
    h1 = _leaky_relu(h1)
    dn = (((1,), (1,)), ((), ()))
    p1_ref[...] = lax.dot_general(h1, W2l_ref[...], dn,
                                  preferred_element_type=jnp.float32)
    r1_ref[...] = lax.dot_general(h1, W2r_ref[...], dn,
                                  preferred_element_type=jnp.float32)


def _stage2_body(q1_ref, r1_ref, b2_ref, W3l_ref, W3r_ref, p2_ref, r2_ref):
    h2 = _leaky_relu(q1_ref[...] + r1_ref[...] + b2_ref[...])
    dn = (((1,), (1,)), ((), ()))
    p2_ref[...] = lax.dot_general(h2, W3l_ref[...], dn,
                                  preferred_element_type=jnp.float32)
    r2_ref[...] = lax.dot_general(h2, W3r_ref[...], dn,
                                  preferred_element_type=jnp.float32)


def _stage3_body(q2_ref, r2_ref, ac_ref, b3_ref, batch_ref, wm_ref, wa_ref,
                 blin_ref, out_ref, sums_ref, cntb_ref):
    i = pl.program_id(0)
    cnt = jnp.maximum(ac_ref[:, 1:2], 1.0)
    h3 = _leaky_relu(q2_ref[...] / cnt + r2_ref[...] + b3_ref[...])
    b = batch_ref[0]
    gid = lax.broadcasted_iota(jnp.int32, (_G, h3.shape[0]), 0)
    onehot = (b == gid).astype(jnp.float32)
    ps = lax.dot_general(onehot, h3, (((1,), (0,)), ((), ())),
                         preferred_element_type=jnp.float32)
    pc = jnp.sum(onehot, axis=1, keepdims=True)

    @pl.when(i == 0)
    def _():
        sums_ref[...] = jnp.zeros_like(sums_ref)
        cntb_ref[...] = jnp.zeros_like(cntb_ref)

    sums_ref[...] += ps
    cntb_ref[:, 0:1] += pc

    @pl.when(i == pl.num_programs(0) - 1)
    def _():
        sums = sums_ref[...]
        cb = jnp.maximum(cntb_ref[:, 0:1], 1.0)
        z = (sums / cb) * wm_ref[...] + sums * wa_ref[...]
        out_ref[...] = jnp.sum(z, axis=1, keepdims=True) + blin_ref[...]


def _row_spec(w):
    return pl.BlockSpec((_RB, w), lambda i: (i, 0))


def _full_spec(shape):
    nd = len(shape)
    return pl.BlockSpec(shape, lambda i: (0,) * nd)


def _tc_params():
    return pltpu.CompilerParams(dimension_semantics=("arbitrary",))


@functools.lru_cache(maxsize=None)
def _stage1_call(N, H):
    grid = (N // _RB,)
    return pl.pallas_call(
        _stage1_body,
        grid=grid,
        in_specs=[_row_spec(1), _row_spec(_L), _full_spec((1, H)),
                  _full_spec((1, H)), _full_spec((1, H)),
                  _full_spec((H, H)), _full_spec((H, H))],
        out_specs=[_row_spec(H), _row_spec(H)],
        out_shape=[jax.ShapeDtypeStruct((N, H), jnp.float32),
                   jax.ShapeDtypeStruct((N, H), jnp.float32)],
        compiler_params=_tc_params(),
    )


@functools.lru_cache(maxsize=None)
def _stage2_call(N, H):
    grid = (N // _RB,)
    return pl.pallas_call(
        _stage2_body,
        grid=grid,
        in_specs=[_row_spec(H), _row_spec(H), _full_spec((1, H)),
                  _full_spec((H, H)), _full_spec((H, H))],
        out_specs=[_row_spec(H), _row_spec(H)],
        out_shape=[jax.ShapeDtypeStruct((N, H), jnp.float32),
                   jax.ShapeDtypeStruct((N, H), jnp.float32)],
        compiler_params=_tc_params(),
    )


@functools.lru_cache(maxsize=None)
def _stage3_call(N, H):
    grid = (N // _RB,)
    return pl.pallas_call(
        _stage3_body,
        grid=grid,
        in_specs=[_row_spec(H), _row_spec(H), _row_spec(_L),
                  _full_spec((1, H)),
                  pl.BlockSpec((1, 1, _RB), lambda i: (i, 0, 0)),
                  _full_spec((1, H)), _full_spec((1, H)),
                  _full_spec((1, 1))],
        out_specs=[_full_spec((_G, 1))],
        out_shape=[jax.ShapeDtypeStruct((_G, 1), jnp.float32)],
        scratch_shapes=[pltpu.VMEM((_G, H), jnp.float32),
                        pltpu.VMEM((_G, 128), jnp.float32)],
        compiler_params=_tc_params(),
    )


def kernel(x, edge_index, batch, W1l, b1, W1r, W2l, b2, W2r, W3l, b3, W3r,
           Wlin, blin):
    N = x.shape[0]
    E = edge_index.shape[1]
    H = W1l.shape[0]
    src = edge_index[0]
    dst = edge_index[1]

    xx = jnp.concatenate(
        [x, jnp.ones((N, 1), jnp.float32), jnp.zeros((N, _L - 2), jnp.float32)],
        axis=1)
    aggcnt = _sc_aggregate(N, _L, E)(xx, src, dst)

    p1, r1 = _stage1_call(N, H)(
        x, aggcnt, W1l.T, W1r.T, b1.reshape(1, H), W2l, W2r)
    q1 = _sc_aggregate(N, H, E)(p1, src, dst)

    p2, r2 = _stage2_call(N, H)(q1, r1, b2.reshape(1, H), W3l, W3r)
    q2 = _sc_aggregate(N, H, E)(p2, src, dst)

    out, = _stage3_call(N, H)(
        q2, r2, aggcnt, b3.reshape(1, H),
        batch.reshape(N // _RB, 1, _RB).astype(jnp.int32),
        Wlin[:, :H], Wlin[:, H:], blin.reshape(1, 1))
    return out


# speedup vs baseline: 1.1037x; 1.1037x over previous
import functools

import jax
import jax.numpy as jnp
from jax import lax
from jax.experimental import pallas as pl
from jax.experimental.pallas import tpu as pltpu
from jax.experimental.pallas import tpu_sc as plsc

_L = 16
_NS = 16
_NC = 2
_G = 64
_NP = 10016
_EP = 160256
_SCH = 512
_CS = 4


def _leaky_relu(v):
    return jnp.where(v >= 0, v, 0.01 * v)


def _sc_params():
    return pltpu.CompilerParams(needs_layout_passes=False,
                                use_tc_tiling_on_sc=False)


def _mesh():
    return plsc.VectorSubcoreMesh(
        core_axis_name="c", subcore_axis_name="s",
        num_cores=_NC, num_subcores=_NS)


_XR = 79
_AR1 = 157


@functools.lru_cache(maxsize=None)
def _sc_pass1(N, E):
    EPT = E // (_NC * _NS)
    NV = EPT // _L

    @functools.partial(
        pl.kernel,
        out_type=jax.ShapeDtypeStruct((_NC * _NS, _AR1, 128), jnp.float32),
        mesh=_mesh(),
        scratch_types=[
            pltpu.VMEM((_XR, 128), jnp.float32),
            pltpu.VMEM((_AR1, 128), jnp.float32),
            pltpu.VMEM((EPT,), jnp.int32),
            pltpu.VMEM((EPT,), jnp.int32),
        ],
        compiler_params=_sc_params(),
    )
    def k(xf_hbm, zero_hbm, src_hbm, dst_hbm, out_hbm, xt, acc, src_v, dst_v):
        c = lax.axis_index("c")
        s = lax.axis_index("s")
        w = s * _NC + c
        one16 = jnp.ones((_L,), jnp.float32)
        one16i = jnp.ones((_L,), jnp.int32)

        pltpu.sync_copy(xf_hbm, xt)
        pltpu.sync_copy(zero_hbm, acc)

        base = w * EPT
        pltpu.sync_copy(src_hbm.at[pl.ds(base, EPT)], src_v)
        pltpu.sync_copy(dst_hbm.at[pl.ds(base, EPT)], dst_v)

        def _vec(v, _):
            s16 = src_v[pl.ds(v * _L, _L)]
            d16 = dst_v[pl.ds(v * _L, _L)]
            vals = plsc.load_gather(xt, [s16 >> 7, s16 & 127])
            rd = d16 >> 6
            cd = (d16 & 63) << 1
            plsc.addupdate_scatter(acc, [rd, cd], vals)
            plsc.addupdate_scatter(acc, [rd, cd + one16i], one16)
            return 0
        lax.fori_loop(0, NV, _vec, 0)

        pltpu.sync_copy(acc, out_hbm.at[w])

    return k


_ARB = _NP * _CS // 128


@functools.lru_cache(maxsize=None)
def _sc_passB(N, E):
    NCHK = E // _SCH
    NV = _SCH // _L

    @functools.partial(
        pl.kernel,
        out_type=jax.ShapeDtypeStruct((_NC * _NS, _ARB, 128), jnp.float32),
        mesh=_mesh(),
        scratch_types=[
            pltpu.VMEM((_ARB, 128), jnp.float32),
            pltpu.VMEM((_ARB, 128), jnp.float32),
            pltpu.VMEM((_SCH,), jnp.int32),
            pltpu.VMEM((_SCH,), jnp.int32),
        ],
        compiler_params=_sc_params(),
    )
    def k(ps_hbm, zero_hbm, src_hbm, dst_hbm, out_hbm, pt, acc, src_v, dst_v):
        c = lax.axis_index("c")
        s = lax.axis_index("s")
        w = s * _NC + c
        one16i = jnp.ones((_L,), jnp.int32)

        pltpu.sync_copy(ps_hbm.at[w], pt)
        pltpu.sync_copy(zero_hbm, acc)

        def _chunk(ch, _):
            pltpu.sync_copy(src_hbm.at[pl.ds(ch * _SCH, _SCH)], src_v)
            pltpu.sync_copy(dst_hbm.at[pl.ds(ch * _SCH, _SCH)], dst_v)

            def _vec(v, __):
                s16 = src_v[pl.ds(v * _L, _L)]
                d16 = dst_v[pl.ds(v * _L, _L)]
                rs = s16 >> 5
                cs = (s16 & 31) << 2
                rd = d16 >> 5
                cd = (d16 & 31) << 2
                for _cc in range(_CS):
                    vals = plsc.load_gather(pt, [rs, cs])
                    plsc.addupdate_scatter(acc, [rd, cd], vals)
                    if _cc + 1 < _CS:
                        cs = cs + one16i
                        cd = cd + one16i
                return 0
            lax.fori_loop(0, NV, _vec, 0)
            return 0
        lax.fori_loop(0, NCHK, _chunk, 0)

        pltpu.sync_copy(acc, out_hbm.at[w])

    return k


_RB = 1000


def _stage1_body(x_ref, pr_ref, w1l_ref, w1r_ref, b1_ref, W2l_ref, W2r_ref,
                 p1_ref, r1_ref, cnt_ref):
    a = jnp.sum(pr_ref[:, :32], axis=1, keepdims=True)
    cnt_ref[...] = jnp.sum(pr_ref[:, 32:], axis=1, keepdims=True)
    h1 = a * w1l_ref[...] + x_ref[...] * w1r_ref[...] + b1_ref[...]
    h1 = _leaky_relu(h1)
    dn = (((1,), (1,)), ((), ()))
    p1_ref[...] = lax.dot_general(h1, W2l_ref[...], dn,
                                  preferred_element_type=jnp.float32)
    r1_ref[...] = lax.dot_general(h1, W2r_ref[...], dn,
                                  preferred_element_type=jnp.float32)


def _stage2_body(q1_ref, r1_ref, b2_ref, W3l_ref, W3r_ref, p2_ref, r2_ref):
    h2 = _leaky_relu(q1_ref[...] + r1_ref[...] + b2_ref[...])
    dn = (((1,), (1,)), ((), ()))
    p2_ref[...] = lax.dot_general(h2, W3l_ref[...], dn,
                                  preferred_element_type=jnp.float32)
    r2_ref[...] = lax.dot_general(h2, W3r_ref[...], dn,
                                  preferred_element_type=jnp.float32)


def _stage3_body(q2_ref, r2_ref, cnt_ref, b3_ref, batch_ref, wm_ref, wa_ref,
                 blin_ref, out_ref, sums_ref, cntb_ref):
    i = pl.program_id(0)
    cnt = jnp.maximum(cnt_ref[...], 1.0)
    h3 = _leaky_relu(q2_ref[...] / cnt + r2_ref[...] + b3_ref[...])
    b = batch_ref[0]
    gid = lax.broadcasted_iota(jnp.int32, (_G, h3.shape[0]), 0)
    onehot = (b == gid).astype(jnp.float32)
    ps = lax.dot_general(onehot, h3, (((1,), (0,)), ((), ())),
                         preferred_element_type=jnp.float32)
    pc = jnp.sum(onehot, axis=1, keepdims=True)

    @pl.when(i == 0)
    def _():
        sums_ref[...] = jnp.zeros_like(sums_ref)
        cntb_ref[...] = jnp.zeros_like(cntb_ref)

    sums_ref[...] += ps
    cntb_ref[:, 0:1] += pc

    @pl.when(i == pl.num_programs(0) - 1)
    def _():
        sums = sums_ref[...]
        cb = jnp.maximum(cntb_ref[:, 0:1], 1.0)
        z = (sums / cb) * wm_ref[...] + sums * wa_ref[...]
        out_ref[...] = jnp.sum(z, axis=1, keepdims=True) + blin_ref[...]


def _row_spec(w):
    return pl.BlockSpec((_RB, w), lambda i: (i, 0))


def _full_spec(shape):
    nd = len(shape)
    return pl.BlockSpec(shape, lambda i: (0,) * nd)


def _tc_params():
    return pltpu.CompilerParams(dimension_semantics=("arbitrary",))


@functools.lru_cache(maxsize=None)
def _stage1_call(N, H):
    grid = (N // _RB,)
    return pl.pallas_call(
        _stage1_body,
        grid=grid,
        in_specs=[_row_spec(1), _row_spec(64), _full_spec((1, H)),
                  _full_spec((1, H)), _full_spec((1, H)),
                  _full_spec((H, H)), _full_spec((H, H))],
        out_specs=[_row_spec(H), _row_spec(H), _row_spec(1)],
        out_shape=[jax.ShapeDtypeStruct((N, H), jnp.float32),
                   jax.ShapeDtypeStruct((N, H), jnp.float32),
                   jax.ShapeDtypeStruct((N, 1), jnp.float32)],
        compiler_params=_tc_params(),
    )


@functools.lru_cache(maxsize=None)
def _stage2_call(N, H):
    grid = (N // _RB,)
    return pl.pallas_call(
        _stage2_body,
        grid=grid,
        in_specs=[_row_spec(H), _row_spec(H), _full_spec((1, H)),
                  _full_spec((H, H)), _full_spec((H, H))],
        out_specs=[_row_spec(H), _row_spec(H)],
        out_shape=[jax.ShapeDtypeStruct((N, H), jnp.float32),
                   jax.ShapeDtypeStruct((N, H), jnp.float32)],
        compiler_params=_tc_params(),
    )


@functools.lru_cache(maxsize=None)
def _stage3_call(N, H):
    grid = (N // _RB,)
    return pl.pallas_call(
        _stage3_body,
        grid=grid,
        in_specs=[_row_spec(H), _row_spec(H), _row_spec(1),
                  _full_spec((1, H)),
                  pl.BlockSpec((1, 1, _RB), lambda i: (i, 0, 0)),
                  _full_spec((1, H)), _full_spec((1, H)),
                  _full_spec((1, 1))],
        out_specs=[_full_spec((_G, 1))],
        out_shape=[jax.ShapeDtypeStruct((_G, 1), jnp.float32)],
        scratch_shapes=[pltpu.VMEM((_G, H), jnp.float32),
                        pltpu.VMEM((_G, 128), jnp.float32)],
        compiler_params=_tc_params(),
    )


def _aggregate(p, srcp, dstp, N, E):
    H = p.shape[1]
    NS32 = _NC * _NS
    ps = p.reshape(N, H // _CS, _CS).transpose(1, 0, 2)
    ps = ps.reshape(H // _CS, N * _CS)
    ps = jnp.pad(ps, ((0, 0), (0, _ARB * 128 - N * _CS)))
    ps = ps.reshape(H // _CS, _ARB, 128)
    zeros = jnp.zeros((_ARB, 128), jnp.float32)
    halves = []
    for h in range(H // (_CS * NS32)):
        qh = _sc_passB(N, E)(ps[h * NS32:(h + 1) * NS32], zeros, srcp, dstp)
        qh = qh.reshape(NS32, _NP, _CS)[:, :N, :]
        halves.append(qh)
    q = jnp.concatenate(halves, axis=0)
    return q.transpose(1, 0, 2).reshape(N, H)


def kernel(x, edge_index, batch, W1l, b1, W1r, W2l, b2, W2r, W3l, b3, W3r,
           Wlin, blin):
    N = x.shape[0]
    E = edge_index.shape[1]
    H = W1l.shape[0]
    pad = _EP - E
    srcp = jnp.pad(edge_index[0], (0, pad))
    dstp = jnp.pad(edge_index[1], (0, pad),
                   constant_values=_NP - 8)

    xf = jnp.pad(x[:, 0], (0, _XR * 128 - N)).reshape(_XR, 128)
    zeros1 = jnp.zeros((_AR1, 128), jnp.float32)
    part = _sc_pass1(N, _EP)(xf, zeros1, srcp, dstp)
    part = part.reshape(32, _AR1 * 128)[:, :_NP * 2].reshape(32, _NP, 2)
    pr = part[:, :N, :].transpose(1, 2, 0).reshape(N, 64)

    p1, r1, cnt = _stage1_call(N, H)(
        x, pr, W1l.T, W1r.T, b1.reshape(1, H), W2l, W2r)
    q1 = _aggregate(p1, srcp, dstp, N, _EP)

    p2, r2 = _stage2_call(N, H)(q1, r1, b2.reshape(1, H), W3l, W3r)
    q2 = _aggregate(p2, srcp, dstp, N, _EP)

    out, = _stage3_call(N, H)(
        q2, r2, cnt, b3.reshape(1, H),
        batch.reshape(N // _RB, 1, _RB).astype(jnp.int32),
        Wlin[:, :H], Wlin[:, H:], blin.reshape(1, 1))
    return out


# speedup vs baseline: 1.2493x; 1.1320x over previous
import functools

import jax
import jax.numpy as jnp
from jax import lax
from jax.experimental import pallas as pl
from jax.experimental.pallas import tpu as pltpu
from jax.experimental.pallas import tpu_sc as plsc

_L = 16
_NS = 16
_NC = 2
_G = 64
_NP = 10016
_EP = 163840
_SCH = 2048
_UNR = 4
_CS = 4


def _leaky_relu(v):
    return jnp.where(v >= 0, v, 0.01 * v)


def _sc_params():
    return pltpu.CompilerParams(needs_layout_passes=False,
                                use_tc_tiling_on_sc=False)


def _mesh():
    return plsc.VectorSubcoreMesh(
        core_axis_name="c", subcore_axis_name="s",
        num_cores=_NC, num_subcores=_NS)


_XR = 79
_AR1 = 157


@functools.lru_cache(maxsize=None)
def _sc_pass1(N, E):
    EPT = E // (_NC * _NS)
    NV = EPT // _L

    @functools.partial(
        pl.kernel,
        out_type=jax.ShapeDtypeStruct((_NC * _NS, _AR1, 128), jnp.float32),
        mesh=_mesh(),
        scratch_types=[
            pltpu.VMEM((_XR, 128), jnp.float32),
            pltpu.VMEM((_AR1, 128), jnp.float32),
            pltpu.VMEM((EPT,), jnp.int32),
            pltpu.VMEM((EPT,), jnp.int32),
        ],
        compiler_params=_sc_params(),
    )
    def k(xf_hbm, zero_hbm, src_hbm, dst_hbm, out_hbm, xt, acc, src_v, dst_v):
        c = lax.axis_index("c")
        s = lax.axis_index("s")
        w = s * _NC + c
        one16 = jnp.ones((_L,), jnp.float32)
        one16i = jnp.ones((_L,), jnp.int32)

        pltpu.sync_copy(xf_hbm, xt)
        pltpu.sync_copy(zero_hbm, acc)

        base = w * EPT
        pltpu.sync_copy(src_hbm.at[pl.ds(base, EPT)], src_v)
        pltpu.sync_copy(dst_hbm.at[pl.ds(base, EPT)], dst_v)

        def _vec(v, _):
            s16 = src_v[pl.ds(v * _L, _L)]
            d16 = dst_v[pl.ds(v * _L, _L)]
            vals = plsc.load_gather(xt, [s16 >> 7, s16 & 127])
            rd = d16 >> 6
            cd = (d16 & 63) << 1
            plsc.addupdate_scatter(acc, [rd, cd], vals)
            plsc.addupdate_scatter(acc, [rd, cd + one16i], one16)
            return 0
        lax.fori_loop(0, NV, _vec, 0)

        pltpu.sync_copy(acc, out_hbm.at[w])

    return k


_ARB = _NP * _CS // 128


@functools.lru_cache(maxsize=None)
def _sc_passB(N, E):
    NCHK = E // _SCH
    NV = _SCH // _L

    @functools.partial(
        pl.kernel,
        out_type=jax.ShapeDtypeStruct((_NC * _NS, _ARB, 128), jnp.float32),
        mesh=_mesh(),
        scratch_types=[
            pltpu.VMEM((_ARB, 128), jnp.float32),
            pltpu.VMEM((_ARB, 128), jnp.float32),
            pltpu.VMEM((_SCH,), jnp.int32),
            pltpu.VMEM((_SCH,), jnp.int32),
            pltpu.VMEM((_SCH,), jnp.int32),
            pltpu.VMEM((_SCH,), jnp.int32),
        ],
        compiler_params=_sc_params(),
    )
    def k(ps_hbm, zero_hbm, rs_hbm, cs_hbm, rd_hbm, cd_hbm, out_hbm,
          pt, acc, rs_v, cs_v, rd_v, cd_v):
        c = lax.axis_index("c")
        s = lax.axis_index("s")
        w = s * _NC + c
        one16i = jnp.ones((_L,), jnp.int32)

        pltpu.sync_copy(ps_hbm.at[w], pt)
        pltpu.sync_copy(zero_hbm, acc)

        def _chunk(ch, _):
            pltpu.sync_copy(rs_hbm.at[pl.ds(ch * _SCH, _SCH)], rs_v)
            pltpu.sync_copy(cs_hbm.at[pl.ds(ch * _SCH, _SCH)], cs_v)
            pltpu.sync_copy(rd_hbm.at[pl.ds(ch * _SCH, _SCH)], rd_v)
            pltpu.sync_copy(cd_hbm.at[pl.ds(ch * _SCH, _SCH)], cd_v)

            def _vec(v, __):
                for u in range(_UNR):
                    o = v * _UNR * _L + u * _L
                    rs = rs_v[pl.ds(o, _L)]
                    cs = cs_v[pl.ds(o, _L)]
                    rd = rd_v[pl.ds(o, _L)]
                    cd = cd_v[pl.ds(o, _L)]
                    for _cc in range(_CS):
                        vals = plsc.load_gather(pt, [rs, cs])
                        plsc.addupdate_scatter(acc, [rd, cd], vals)
                        if _cc + 1 < _CS:
                            cs = cs + one16i
                            cd = cd + one16i
                return 0
            lax.fori_loop(0, NV // _UNR, _vec, 0)
            return 0
        lax.fori_loop(0, NCHK, _chunk, 0)

        pltpu.sync_copy(acc, out_hbm.at[w])

    return k


_RB = 1000


def _stage1_body(x_ref, pr_ref, w1l_ref, w1r_ref, b1_ref, W2l_ref, W2r_ref,
                 p1_ref, r1_ref, cnt_ref):
    a = jnp.sum(pr_ref[:, :32], axis=1, keepdims=True)
    cnt_ref[...] = jnp.sum(pr_ref[:, 32:], axis=1, keepdims=True)
    h1 = a * w1l_ref[...] + x_ref[...] * w1r_ref[...] + b1_ref[...]
    h1 = _leaky_relu(h1)
    dn = (((1,), (1,)), ((), ()))
    p1_ref[...] = lax.dot_general(h1, W2l_ref[...], dn,
                                  preferred_element_type=jnp.float32)
    r1_ref[...] = lax.dot_general(h1, W2r_ref[...], dn,
                                  preferred_element_type=jnp.float32)


def _stage2_body(q1_ref, r1_ref, b2_ref, W3l_ref, W3r_ref, p2_ref, r2_ref):
    h2 = _leaky_relu(q1_ref[...] + r1_ref[...] + b2_ref[...])
    dn = (((1,), (1,)), ((), ()))
    p2_ref[...] = lax.dot_general(h2, W3l_ref[...], dn,
                                  preferred_element_type=jnp.float32)
    r2_ref[...] = lax.dot_general(h2, W3r_ref[...], dn,
                                  preferred_element_type=jnp.float32)


def _stage3_body(q2_ref, r2_ref, cnt_ref, b3_ref, batch_ref, wm_ref, wa_ref,
                 blin_ref, out_ref, sums_ref, cntb_ref):
    i = pl.program_id(0)
    cnt = jnp.maximum(cnt_ref[...], 1.0)
    h3 = _leaky_relu(q2_ref[...] / cnt + r2_ref[...] + b3_ref[...])
    b = batch_ref[0]
    gid = lax.broadcasted_iota(jnp.int32, (_G, h3.shape[0]), 0)
    onehot = (b == gid).astype(jnp.float32)
    ps = lax.dot_general(onehot, h3, (((1,), (0,)), ((), ())),
                         preferred_element_type=jnp.float32)
    pc = jnp.sum(onehot, axis=1, keepdims=True)

    @pl.when(i == 0)
    def _():
        sums_ref[...] = jnp.zeros_like(sums_ref)
        cntb_ref[...] = jnp.zeros_like(cntb_ref)

    sums_ref[...] += ps
    cntb_ref[:, 0:1] += pc

    @pl.when(i == pl.num_programs(0) - 1)
    def _():
        sums = sums_ref[...]
        cb = jnp.maximum(cntb_ref[:, 0:1], 1.0)
        z = (sums / cb) * wm_ref[...] + sums * wa_ref[...]
        out_ref[...] = jnp.sum(z, axis=1, keepdims=True) + blin_ref[...]


def _row_spec(w):
    return pl.BlockSpec((_RB, w), lambda i: (i, 0))


def _full_spec(shape):
    nd = len(shape)
    return pl.BlockSpec(shape, lambda i: (0,) * nd)


def _tc_params():
    return pltpu.CompilerParams(dimension_semantics=("arbitrary",))


@functools.lru_cache(maxsize=None)
def _stage1_call(N, H):
    grid = (N // _RB,)
    return pl.pallas_call(
        _stage1_body,
        grid=grid,
        in_specs=[_row_spec(1), _row_spec(64), _full_spec((1, H)),
                  _full_spec((1, H)), _full_spec((1, H)),
                  _full_spec((H, H)), _full_spec((H, H))],
        out_specs=[_row_spec(H), _row_spec(H), _row_spec(1)],
        out_shape=[jax.ShapeDtypeStruct((N, H), jnp.float32),
                   jax.ShapeDtypeStruct((N, H), jnp.float32),
                   jax.ShapeDtypeStruct((N, 1), jnp.float32)],
        compiler_params=_tc_params(),
    )


@functools.lru_cache(maxsize=None)
def _stage2_call(N, H):
    grid = (N // _RB,)
    return pl.pallas_call(
        _stage2_body,
        grid=grid,
        in_specs=[_row_spec(H), _row_spec(H), _full_spec((1, H)),
                  _full_spec((H, H)), _full_spec((H, H))],
        out_specs=[_row_spec(H), _row_spec(H)],
        out_shape=[jax.ShapeDtypeStruct((N, H), jnp.float32),
                   jax.ShapeDtypeStruct((N, H), jnp.float32)],
        compiler_params=_tc_params(),
    )


@functools.lru_cache(maxsize=None)
def _stage3_call(N, H):
    grid = (N // _RB,)
    return pl.pallas_call(
        _stage3_body,
        grid=grid,
        in_specs=[_row_spec(H), _row_spec(H), _row_spec(1),
                  _full_spec((1, H)),
                  pl.BlockSpec((1, 1, _RB), lambda i: (i, 0, 0)),
                  _full_spec((1, H)), _full_spec((1, H)),
                  _full_spec((1, 1))],
        out_specs=[_full_spec((_G, 1))],
        out_shape=[jax.ShapeDtypeStruct((_G, 1), jnp.float32)],
        scratch_shapes=[pltpu.VMEM((_G, H), jnp.float32),
                        pltpu.VMEM((_G, 128), jnp.float32)],
        compiler_params=_tc_params(),
    )


def _aggregate(p, eidx, N, E):
    H = p.shape[1]
    NS32 = _NC * _NS
    ps = p.reshape(N, H // _CS, _CS).transpose(1, 0, 2)
    ps = ps.reshape(H // _CS, N * _CS)
    ps = jnp.pad(ps, ((0, 0), (0, _ARB * 128 - N * _CS)))
    ps = ps.reshape(H // _CS, _ARB, 128)
    zeros = jnp.zeros((_ARB, 128), jnp.float32)
    halves = []
    for h in range(H // (_CS * NS32)):
        qh = _sc_passB(N, E)(ps[h * NS32:(h + 1) * NS32], zeros, *eidx)
        qh = qh.reshape(NS32, _NP, _CS)[:, :N, :]
        halves.append(qh)
    q = jnp.concatenate(halves, axis=0)
    return q.transpose(1, 0, 2).reshape(N, H)


def kernel(x, edge_index, batch, W1l, b1, W1r, W2l, b2, W2r, W3l, b3, W3r,
           Wlin, blin):
    N = x.shape[0]
    E = edge_index.shape[1]
    H = W1l.shape[0]
    pad = _EP - E
    srcp = jnp.pad(edge_index[0], (0, pad))
    dstp = jnp.pad(edge_index[1], (0, pad),
                   constant_values=_NP - 8)

    xf = jnp.pad(x[:, 0], (0, _XR * 128 - N)).reshape(_XR, 128)
    zeros1 = jnp.zeros((_AR1, 128), jnp.float32)
    part = _sc_pass1(N, _EP)(xf, zeros1, srcp, dstp)
    part = part.reshape(32, _AR1 * 128)[:, :_NP * 2].reshape(32, _NP, 2)
    pr = part[:, :N, :].transpose(1, 2, 0).reshape(N, 64)

    p1, r1, cnt = _stage1_call(N, H)(
        x, pr, W1l.T, W1r.T, b1.reshape(1, H), W2l, W2r)
    eidx = (srcp >> 5, (srcp & 31) << 2, dstp >> 5, (dstp & 31) << 2)
    q1 = _aggregate(p1, eidx, N, _EP)

    p2, r2 = _stage2_call(N, H)(q1, r1, b2.reshape(1, H), W3l, W3r)
    q2 = _aggregate(p2, eidx, N, _EP)

    out, = _stage3_call(N, H)(
        q2, r2, cnt, b3.reshape(1, H),
        batch.reshape(N // _RB, 1, _RB).astype(jnp.int32),
        Wlin[:, :H], Wlin[:, H:], blin.reshape(1, 1))
    return out


# speedup vs baseline: 1.5584x; 1.2474x over previous
import functools

import jax
import jax.numpy as jnp
from jax import lax
from jax.experimental import pallas as pl
from jax.experimental.pallas import tpu as pltpu
from jax.experimental.pallas import tpu_sc as plsc

_L = 16
_NS = 16
_NC = 2
_G = 64
_NP = 10016
_EP = 163840
_SCH = 2048
_UNR = 4
_CS = 4


def _leaky_relu(v):
    return jnp.where(v >= 0, v, 0.01 * v)


def _sc_params():
    return pltpu.CompilerParams(needs_layout_passes=False,
                                use_tc_tiling_on_sc=False)


def _mesh():
    return plsc.VectorSubcoreMesh(
        core_axis_name="c", subcore_axis_name="s",
        num_cores=_NC, num_subcores=_NS)


_XR = 79
_AR1 = 157


@functools.lru_cache(maxsize=None)
def _sc_pass1(N, E):
    EPT = E // (_NC * _NS)
    NV = EPT // _L

    @functools.partial(
        pl.kernel,
        out_type=jax.ShapeDtypeStruct((_NC * _NS, _AR1 * 128), jnp.float32),
        mesh=_mesh(),
        scratch_types=[
            pltpu.VMEM((_XR * 128,), jnp.float32),
            pltpu.VMEM((_AR1 * 128,), jnp.float32),
            pltpu.VMEM((EPT,), jnp.int32),
            pltpu.VMEM((EPT,), jnp.int32),
        ],
        compiler_params=_sc_params(),
    )
    def k(xf_hbm, zero_hbm, src_hbm, dst2_hbm, out_hbm, xt, acc, src_v, dst_v):
        c = lax.axis_index("c")
        s = lax.axis_index("s")
        w = s * _NC + c
        one16 = jnp.ones((_L,), jnp.float32)
        one16i = jnp.ones((_L,), jnp.int32)

        pltpu.sync_copy(xf_hbm, xt)
        pltpu.sync_copy(zero_hbm, acc)

        base = w * EPT
        pltpu.sync_copy(src_hbm.at[pl.ds(base, EPT)], src_v)
        pltpu.sync_copy(dst2_hbm.at[pl.ds(base, EPT)], dst_v)

        def _vec(v, _):
            s16 = src_v[pl.ds(v * _L, _L)]
            f16 = dst_v[pl.ds(v * _L, _L)]
            vals = plsc.load_gather(xt, [s16])
            plsc.addupdate_scatter(acc, [f16], vals)
            plsc.addupdate_scatter(acc, [f16 + one16i], one16)
            return 0
        lax.fori_loop(0, NV, _vec, 0)

        pltpu.sync_copy(acc, out_hbm.at[w])

    return k


_ARB = _NP * _CS // 128


@functools.lru_cache(maxsize=None)
def _sc_passB(N, E):
    NCHK = E // _SCH
    NV = _SCH // _L

    @functools.partial(
        pl.kernel,
        out_type=jax.ShapeDtypeStruct((_NC * _NS, _ARB * 128), jnp.float32),
        mesh=_mesh(),
        scratch_types=[
            pltpu.VMEM((_ARB * 128,), jnp.float32),
            pltpu.VMEM((_ARB * 128,), jnp.float32),
            pltpu.VMEM((_SCH,), jnp.int32),
            pltpu.VMEM((_SCH,), jnp.int32),
        ],
        compiler_params=_sc_params(),
    )
    def k(ps_hbm, zero_hbm, fs_hbm, fd_hbm, out_hbm, pt, acc, fs_v, fd_v):
        c = lax.axis_index("c")
        s = lax.axis_index("s")
        w = s * _NC + c
        one16i = jnp.ones((_L,), jnp.int32)

        pltpu.sync_copy(ps_hbm.at[w], pt)
        pltpu.sync_copy(zero_hbm, acc)

        def _chunk(ch, _):
            pltpu.sync_copy(fs_hbm.at[pl.ds(ch * _SCH, _SCH)], fs_v)
            pltpu.sync_copy(fd_hbm.at[pl.ds(ch * _SCH, _SCH)], fd_v)

            def _vec(v, __):
                for u in range(_UNR):
                    o = v * _UNR * _L + u * _L
                    fs = fs_v[pl.ds(o, _L)]
                    fd = fd_v[pl.ds(o, _L)]
                    for _cc in range(_CS):
                        vals = plsc.load_gather(pt, [fs])
                        plsc.addupdate_scatter(acc, [fd], vals)
                        if _cc + 1 < _CS:
                            fs = fs + one16i
                            fd = fd + one16i
                return 0
            lax.fori_loop(0, NV // _UNR, _vec, 0)
            return 0
        lax.fori_loop(0, NCHK, _chunk, 0)

        pltpu.sync_copy(acc, out_hbm.at[w])

    return k


_RB = 1000


def _stage1_body(x_ref, pr_ref, w1l_ref, w1r_ref, b1_ref, W2l_ref, W2r_ref,
                 p1_ref, r1_ref, cnt_ref):
    a = jnp.sum(pr_ref[:, :32], axis=1, keepdims=True)
    cnt_ref[...] = jnp.sum(pr_ref[:, 32:], axis=1, keepdims=True)
    h1 = a * w1l_ref[...] + x_ref[...] * w1r_ref[...] + b1_ref[...]
    h1 = _leaky_relu(h1)
    dn = (((1,), (1,)), ((), ()))
    p1_ref[...] = lax.dot_general(h1, W2l_ref[...], dn,
                                  preferred_element_type=jnp.float32)
    r1_ref[...] = lax.dot_general(h1, W2r_ref[...], dn,
                                  preferred_element_type=jnp.float32)


def _stage2_body(q1_ref, r1_ref, b2_ref, W3l_ref, W3r_ref, p2_ref, r2_ref):
    h2 = _leaky_relu(q1_ref[...] + r1_ref[...] + b2_ref[...])
    dn = (((1,), (1,)), ((), ()))
    p2_ref[...] = lax.dot_general(h2, W3l_ref[...], dn,
                                  preferred_element_type=jnp.float32)
    r2_ref[...] = lax.dot_general(h2, W3r_ref[...], dn,
                                  preferred_element_type=jnp.float32)


def _stage3_body(q2_ref, r2_ref, cnt_ref, b3_ref, batch_ref, wm_ref, wa_ref,
                 blin_ref, out_ref, sums_ref, cntb_ref):
    i = pl.program_id(0)
    cnt = jnp.maximum(cnt_ref[...], 1.0)
    h3 = _leaky_relu(q2_ref[...] / cnt + r2_ref[...] + b3_ref[...])
    b = batch_ref[0]
    gid = lax.broadcasted_iota(jnp.int32, (_G, h3.shape[0]), 0)
    onehot = (b == gid).astype(jnp.float32)
    ps = lax.dot_general(onehot, h3, (((1,), (0,)), ((), ())),
                         preferred_element_type=jnp.float32)
    pc = jnp.sum(onehot, axis=1, keepdims=True)

    @pl.when(i == 0)
    def _():
        sums_ref[...] = jnp.zeros_like(sums_ref)
        cntb_ref[...] = jnp.zeros_like(cntb_ref)

    sums_ref[...] += ps
    cntb_ref[:, 0:1] += pc

    @pl.when(i == pl.num_programs(0) - 1)
    def _():
        sums = sums_ref[...]
        cb = jnp.maximum(cntb_ref[:, 0:1], 1.0)
        z = (sums / cb) * wm_ref[...] + sums * wa_ref[...]
        out_ref[...] = jnp.sum(z, axis=1, keepdims=True) + blin_ref[...]


def _row_spec(w):
    return pl.BlockSpec((_RB, w), lambda i: (i, 0))


def _full_spec(shape):
    nd = len(shape)
    return pl.BlockSpec(shape, lambda i: (0,) * nd)


def _tc_params():
    return pltpu.CompilerParams(dimension_semantics=("arbitrary",))


@functools.lru_cache(maxsize=None)
def _stage1_call(N, H):
    grid = (N // _RB,)
    return pl.pallas_call(
        _stage1_body,
        grid=grid,
        in_specs=[_row_spec(1), _row_spec(64), _full_spec((1, H)),
                  _full_spec((1, H)), _full_spec((1, H)),
                  _full_spec((H, H)), _full_spec((H, H))],
        out_specs=[_row_spec(H), _row_spec(H), _row_spec(1)],
        out_shape=[jax.ShapeDtypeStruct((N, H), jnp.float32),
                   jax.ShapeDtypeStruct((N, H), jnp.float32),
                   jax.ShapeDtypeStruct((N, 1), jnp.float32)],
        compiler_params=_tc_params(),
    )


@functools.lru_cache(maxsize=None)
def _stage2_call(N, H):
    grid = (N // _RB,)
    return pl.pallas_call(
        _stage2_body,
        grid=grid,
        in_specs=[_row_spec(H), _row_spec(H), _full_spec((1, H)),
                  _full_spec((H, H)), _full_spec((H, H))],
        out_specs=[_row_spec(H), _row_spec(H)],
        out_shape=[jax.ShapeDtypeStruct((N, H), jnp.float32),
                   jax.ShapeDtypeStruct((N, H), jnp.float32)],
        compiler_params=_tc_params(),
    )


@functools.lru_cache(maxsize=None)
def _stage3_call(N, H):
    grid = (N // _RB,)
    return pl.pallas_call(
        _stage3_body,
        grid=grid,
        in_specs=[_row_spec(H), _row_spec(H), _row_spec(1),
                  _full_spec((1, H)),
                  pl.BlockSpec((1, 1, _RB), lambda i: (i, 0, 0)),
                  _full_spec((1, H)), _full_spec((1, H)),
                  _full_spec((1, 1))],
        out_specs=[_full_spec((_G, 1))],
        out_shape=[jax.ShapeDtypeStruct((_G, 1), jnp.float32)],
        scratch_shapes=[pltpu.VMEM((_G, H), jnp.float32),
                        pltpu.VMEM((_G, 128), jnp.float32)],
        compiler_params=_tc_params(),
    )


def _aggregate(p, eidx, N, E):
    H = p.shape[1]
    NS32 = _NC * _NS
    ps = p.reshape(N, H // _CS, _CS).transpose(1, 0, 2)
    ps = ps.reshape(H // _CS, N * _CS)
    ps = jnp.pad(ps, ((0, 0), (0, _ARB * 128 - N * _CS)))
    zeros = jnp.zeros((_ARB * 128,), jnp.float32)
    halves = []
    for h in range(H // (_CS * NS32)):
        qh = _sc_passB(N, E)(ps[h * NS32:(h + 1) * NS32], zeros, *eidx)
        qh = qh.reshape(NS32, _NP, _CS)[:, :N, :]
        halves.append(qh)
    q = jnp.concatenate(halves, axis=0)
    return q.transpose(1, 0, 2).reshape(N, H)


def kernel(x, edge_index, batch, W1l, b1, W1r, W2l, b2, W2r, W3l, b3, W3r,
           Wlin, blin):
    N = x.shape[0]
    E = edge_index.shape[1]
    H = W1l.shape[0]
    pad = _EP - E
    srcp = jnp.pad(edge_index[0], (0, pad))
    dstp = jnp.pad(edge_index[1], (0, pad),
                   constant_values=_NP - 8)

    xf = jnp.pad(x[:, 0], (0, _XR * 128 - N))
    zeros1 = jnp.zeros((_AR1 * 128,), jnp.float32)
    part = _sc_pass1(N, _EP)(xf, zeros1, srcp, dstp * 2)
    part = part[:, :_NP * 2].reshape(32, _NP, 2)
    pr = part[:, :N, :].transpose(1, 2, 0).reshape(N, 64)

    p1, r1, cnt = _stage1_call(N, H)(
        x, pr, W1l.T, W1r.T, b1.reshape(1, H), W2l, W2r)
    eidx = (srcp * 4, dstp * 4)
    q1 = _aggregate(p1, eidx, N, _EP)

    p2, r2 = _stage2_call(N, H)(q1, r1, b2.reshape(1, H), W3l, W3r)
    q2 = _aggregate(p2, eidx, N, _EP)

    out, = _stage3_call(N, H)(
        q2, r2, cnt, b3.reshape(1, H),
        batch.reshape(N // _RB, 1, _RB).astype(jnp.int32),
        Wlin[:, :H], Wlin[:, H:], blin.reshape(1, 1))
    return out


# speedup vs baseline: 2.2889x; 1.4688x over previous
import functools

import jax
import jax.numpy as jnp
from jax import lax
from jax.experimental import pallas as pl
from jax.experimental.pallas import tpu as pltpu
from jax.experimental.pallas import tpu_sc as plsc

_L = 16
_NS = 16
_NC = 2
_G = 64
_NP = 10016
_EP = 163840
_SCH = 2048
_UNR = 4
_CS = 4


def _leaky_relu(v):
    return jnp.where(v >= 0, v, 0.01 * v)


def _sc_params():
    return pltpu.CompilerParams(needs_layout_passes=False,
                                use_tc_tiling_on_sc=False)


def _mesh():
    return plsc.VectorSubcoreMesh(
        core_axis_name="c", subcore_axis_name="s",
        num_cores=_NC, num_subcores=_NS)


_XR = 79
_AR1 = 157


@functools.lru_cache(maxsize=None)
def _sc_pass1(N, E):
    EPT = E // (_NC * _NS)
    NV = EPT // _L

    @functools.partial(
        pl.kernel,
        out_type=jax.ShapeDtypeStruct((_NC * _NS, _AR1 * 128), jnp.float32),
        mesh=_mesh(),
        scratch_types=[
            pltpu.VMEM((_XR * 128,), jnp.float32),
            pltpu.VMEM((_AR1 * 128,), jnp.float32),
            pltpu.VMEM((EPT,), jnp.int32),
            pltpu.VMEM((EPT,), jnp.int32),
        ],
        compiler_params=_sc_params(),
    )
    def k(xf_hbm, zero_hbm, src_hbm, dst2_hbm, out_hbm, xt, acc, src_v, dst_v):
        c = lax.axis_index("c")
        s = lax.axis_index("s")
        w = s * _NC + c
        one16 = jnp.ones((_L,), jnp.float32)
        one16i = jnp.ones((_L,), jnp.int32)

        pltpu.sync_copy(xf_hbm, xt)
        pltpu.sync_copy(zero_hbm, acc)

        base = w * EPT
        pltpu.sync_copy(src_hbm.at[pl.ds(base, EPT)], src_v)
        pltpu.sync_copy(dst2_hbm.at[pl.ds(base, EPT)], dst_v)

        def _vec(v, _):
            s16 = src_v[pl.ds(v * _L, _L)]
            f16 = dst_v[pl.ds(v * _L, _L)]
            vals = plsc.load_gather(xt, [s16])
            plsc.addupdate_scatter(acc, [f16], vals)
            plsc.addupdate_scatter(acc, [f16 + one16i], one16)
            return 0
        lax.fori_loop(0, NV, _vec, 0)

        pltpu.sync_copy(acc, out_hbm.at[w])

    return k


_ARB = _NP * _CS // 128


@functools.lru_cache(maxsize=None)
def _sc_passB(N, E):
    NCHK = E // _SCH
    NV = _SCH // _L

    @functools.partial(
        pl.kernel,
        out_type=jax.ShapeDtypeStruct((_NC * _NS, _ARB * 128), jnp.float32),
        mesh=_mesh(),
        scratch_types=[
            pltpu.VMEM((_ARB * 128,), jnp.float32),
            pltpu.VMEM((_ARB * 128,), jnp.float32),
            pltpu.VMEM((_SCH,), jnp.int32),
            pltpu.VMEM((_SCH,), jnp.int32),
        ],
        compiler_params=_sc_params(),
    )
    def k(ps_hbm, zero_hbm, fs_hbm, fd_hbm, out_hbm, pt, acc, fs_v, fd_v):
        c = lax.axis_index("c")
        s = lax.axis_index("s")
        w = s * _NC + c
        one16i = jnp.ones((_L,), jnp.int32)

        pltpu.sync_copy(ps_hbm.at[w], pt)
        pltpu.sync_copy(zero_hbm, acc)

        def _chunk(ch, _):
            pltpu.sync_copy(fs_hbm.at[pl.ds(ch * _SCH, _SCH)], fs_v)
            pltpu.sync_copy(fd_hbm.at[pl.ds(ch * _SCH, _SCH)], fd_v)

            def _vec(v, __):
                gathered = []
                for u in range(_UNR):
                    o = v * _UNR * _L + u * _L
                    fs = fs_v[pl.ds(o, _L)]
                    fd = fd_v[pl.ds(o, _L)]
                    for _cc in range(_CS):
                        gathered.append((fd, plsc.load_gather(pt, [fs])))
                        if _cc + 1 < _CS:
                            fs = fs + one16i
                            fd = fd + one16i
                for fd, vals in gathered:
                    plsc.addupdate_scatter(acc, [fd], vals)
                return 0
            lax.fori_loop(0, NV // _UNR, _vec, 0)
            return 0
        lax.fori_loop(0, NCHK, _chunk, 0)

        pltpu.sync_copy(acc, out_hbm.at[w])

    return k


_RB = 1000


def _stage1_body(x_ref, pr_ref, w1l_ref, w1r_ref, b1_ref, W2l_ref, W2r_ref,
                 p1_ref, r1_ref, cnt_ref):
    a = jnp.sum(pr_ref[:, :32], axis=1, keepdims=True)
    cnt_ref[...] = jnp.sum(pr_ref[:, 32:], axis=1, keepdims=True)
    h1 = a * w1l_ref[...] + x_ref[...] * w1r_ref[...] + b1_ref[...]
    h1 = _leaky_relu(h1)
    dn = (((1,), (1,)), ((), ()))
    p1_ref[...] = lax.dot_general(h1, W2l_ref[...], dn,
                                  preferred_element_type=jnp.float32)
    r1_ref[...] = lax.dot_general(h1, W2r_ref[...], dn,
                                  preferred_element_type=jnp.float32)


def _stage2_body(q1_ref, r1_ref, b2_ref, W3l_ref, W3r_ref, p2_ref, r2_ref):
    h2 = _leaky_relu(q1_ref[...] + r1_ref[...] + b2_ref[...])
    dn = (((1,), (1,)), ((), ()))
    p2_ref[...] = lax.dot_general(h2, W3l_ref[...], dn,
                                  preferred_element_type=jnp.float32)
    r2_ref[...] = lax.dot_general(h2, W3r_ref[...], dn,
                                  preferred_element_type=jnp.float32)


def _stage3_body(q2_ref, r2_ref, cnt_ref, b3_ref, batch_ref, wm_ref, wa_ref,
                 blin_ref, out_ref, sums_ref, cntb_ref):
    i = pl.program_id(0)
    cnt = jnp.maximum(cnt_ref[...], 1.0)
    h3 = _leaky_relu(q2_ref[...] / cnt + r2_ref[...] + b3_ref[...])
    b = batch_ref[0]
    gid = lax.broadcasted_iota(jnp.int32, (_G, h3.shape[0]), 0)
    onehot = (b == gid).astype(jnp.float32)
    ps = lax.dot_general(onehot, h3, (((1,), (0,)), ((), ())),
                         preferred_element_type=jnp.float32)
    pc = jnp.sum(onehot, axis=1, keepdims=True)

    @pl.when(i == 0)
    def _():
        sums_ref[...] = jnp.zeros_like(sums_ref)
        cntb_ref[...] = jnp.zeros_like(cntb_ref)

    sums_ref[...] += ps
    cntb_ref[:, 0:1] += pc

    @pl.when(i == pl.num_programs(0) - 1)
    def _():
        sums = sums_ref[...]
        cb = jnp.maximum(cntb_ref[:, 0:1], 1.0)
        z = (sums / cb) * wm_ref[...] + sums * wa_ref[...]
        out_ref[...] = jnp.sum(z, axis=1, keepdims=True) + blin_ref[...]


def _row_spec(w):
    return pl.BlockSpec((_RB, w), lambda i: (i, 0))


def _full_spec(shape):
    nd = len(shape)
    return pl.BlockSpec(shape, lambda i: (0,) * nd)


def _tc_params():
    return pltpu.CompilerParams(dimension_semantics=("arbitrary",))


@functools.lru_cache(maxsize=None)
def _stage1_call(N, H):
    grid = (N // _RB,)
    return pl.pallas_call(
        _stage1_body,
        grid=grid,
        in_specs=[_row_spec(1), _row_spec(64), _full_spec((1, H)),
                  _full_spec((1, H)), _full_spec((1, H)),
                  _full_spec((H, H)), _full_spec((H, H))],
        out_specs=[_row_spec(H), _row_spec(H), _row_spec(1)],
        out_shape=[jax.ShapeDtypeStruct((N, H), jnp.float32),
                   jax.ShapeDtypeStruct((N, H), jnp.float32),
                   jax.ShapeDtypeStruct((N, 1), jnp.float32)],
        compiler_params=_tc_params(),
    )


@functools.lru_cache(maxsize=None)
def _stage2_call(N, H):
    grid = (N // _RB,)
    return pl.pallas_call(
        _stage2_body,
        grid=grid,
        in_specs=[_row_spec(H), _row_spec(H), _full_spec((1, H)),
                  _full_spec((H, H)), _full_spec((H, H))],
        out_specs=[_row_spec(H), _row_spec(H)],
        out_shape=[jax.ShapeDtypeStruct((N, H), jnp.float32),
                   jax.ShapeDtypeStruct((N, H), jnp.float32)],
        compiler_params=_tc_params(),
    )


@functools.lru_cache(maxsize=None)
def _stage3_call(N, H):
    grid = (N // _RB,)
    return pl.pallas_call(
        _stage3_body,
        grid=grid,
        in_specs=[_row_spec(H), _row_spec(H), _row_spec(1),
                  _full_spec((1, H)),
                  pl.BlockSpec((1, 1, _RB), lambda i: (i, 0, 0)),
                  _full_spec((1, H)), _full_spec((1, H)),
                  _full_spec((1, 1))],
        out_specs=[_full_spec((_G, 1))],
        out_shape=[jax.ShapeDtypeStruct((_G, 1), jnp.float32)],
        scratch_shapes=[pltpu.VMEM((_G, H), jnp.float32),
                        pltpu.VMEM((_G, 128), jnp.float32)],
        compiler_params=_tc_params(),
    )


def _aggregate(p, eidx, N, E):
    H = p.shape[1]
    NS32 = _NC * _NS
    ps = p.reshape(N, H // _CS, _CS).transpose(1, 0, 2)
    ps = ps.reshape(H // _CS, N * _CS)
    ps = jnp.pad(ps, ((0, 0), (0, _ARB * 128 - N * _CS)))
    zeros = jnp.zeros((_ARB * 128,), jnp.float32)
    halves = []
    for h in range(H // (_CS * NS32)):
        qh = _sc_passB(N, E)(ps[h * NS32:(h + 1) * NS32], zeros, *eidx)
        qh = qh.reshape(NS32, _NP, _CS)[:, :N, :]
        halves.append(qh)
    q = jnp.concatenate(halves, axis=0)
    return q.transpose(1, 0, 2).reshape(N, H)


def kernel(x, edge_index, batch, W1l, b1, W1r, W2l, b2, W2r, W3l, b3, W3r,
           Wlin, blin):
    N = x.shape[0]
    E = edge_index.shape[1]
    H = W1l.shape[0]
    pad = _EP - E
    srcp = jnp.pad(edge_index[0], (0, pad))
    dstp = jnp.pad(edge_index[1], (0, pad),
                   constant_values=_NP - 8)

    xf = jnp.pad(x[:, 0], (0, _XR * 128 - N))
    zeros1 = jnp.zeros((_AR1 * 128,), jnp.float32)
    part = _sc_pass1(N, _EP)(xf, zeros1, srcp, dstp * 2)
    part = part[:, :_NP * 2].reshape(32, _NP, 2)
    pr = part[:, :N, :].transpose(1, 2, 0).reshape(N, 64)

    p1, r1, cnt = _stage1_call(N, H)(
        x, pr, W1l.T, W1r.T, b1.reshape(1, H), W2l, W2r)
    eidx = (srcp * 4, dstp * 4)
    q1 = _aggregate(p1, eidx, N, _EP)

    p2, r2 = _stage2_call(N, H)(q1, r1, b2.reshape(1, H), W3l, W3r)
    q2 = _aggregate(p2, eidx, N, _EP)

    out, = _stage3_call(N, H)(
        q2, r2, cnt, b3.reshape(1, H),
        batch.reshape(N // _RB, 1, _RB).astype(jnp.int32),
        Wlin[:, :H], Wlin[:, H:], blin.reshape(1, 1))
    return out


# speedup vs baseline: 2.2903x; 1.0006x over previous
import functools

import jax
import jax.numpy as jnp
from jax import lax
from jax.experimental import pallas as pl
from jax.experimental.pallas import tpu as pltpu
from jax.experimental.pallas import tpu_sc as plsc

_L = 16
_NS = 16
_NC = 2
_G = 64
_NP = 10016
_EP = 163840
_SCH = 2048
_UNR = 4
_CS = 4


def _leaky_relu(v):
    return jnp.where(v >= 0, v, 0.01 * v)


def _sc_params():
    return pltpu.CompilerParams(needs_layout_passes=False,
                                use_tc_tiling_on_sc=False)


def _mesh():
    return plsc.VectorSubcoreMesh(
        core_axis_name="c", subcore_axis_name="s",
        num_cores=_NC, num_subcores=_NS)


_XR = 79
_AR1 = 157


@functools.lru_cache(maxsize=None)
def _sc_pass1(N, E):
    EPT = E // (_NC * _NS)
    NV = EPT // _L

    @functools.partial(
        pl.kernel,
        out_type=jax.ShapeDtypeStruct((_NC * _NS, _AR1 * 128), jnp.float32),
        mesh=_mesh(),
        scratch_types=[
            pltpu.VMEM((_XR * 128,), jnp.float32),
            pltpu.VMEM((_AR1 * 128,), jnp.float32),
            pltpu.VMEM((EPT,), jnp.int32),
            pltpu.VMEM((EPT,), jnp.int32),
        ],
        compiler_params=_sc_params(),
    )
    def k(xf_hbm, zero_hbm, src_hbm, dst2_hbm, out_hbm, xt, acc, src_v, dst_v):
        c = lax.axis_index("c")
        s = lax.axis_index("s")
        w = s * _NC + c
        one16 = jnp.ones((_L,), jnp.float32)
        one16i = jnp.ones((_L,), jnp.int32)

        pltpu.sync_copy(xf_hbm, xt)
        pltpu.sync_copy(zero_hbm, acc)

        base = w * EPT
        pltpu.sync_copy(src_hbm.at[pl.ds(base, EPT)], src_v)
        pltpu.sync_copy(dst2_hbm.at[pl.ds(base, EPT)], dst_v)

        def _vec(v, _):
            s16 = src_v[pl.ds(v * _L, _L)]
            f16 = dst_v[pl.ds(v * _L, _L)]
            vals = plsc.load_gather(xt, [s16])
            plsc.addupdate_scatter(acc, [f16], vals)
            plsc.addupdate_scatter(acc, [f16 + one16i], one16)
            return 0
        lax.fori_loop(0, NV, _vec, 0)

        pltpu.sync_copy(acc, out_hbm.at[w])

    return k


_ARB = _NP * _CS // 128


@functools.lru_cache(maxsize=None)
def _sc_passB(N, E):
    NCHK = E // _SCH
    NV = _SCH // _L

    @functools.partial(
        pl.kernel,
        out_type=jax.ShapeDtypeStruct((_NC * _NS, _ARB * 128), jnp.float32),
        mesh=_mesh(),
        scratch_types=[
            pltpu.VMEM((_ARB * 128,), jnp.float32),
            pltpu.VMEM((_ARB * 128,), jnp.float32),
            pltpu.VMEM((_SCH,), jnp.int32),
            pltpu.VMEM((_SCH,), jnp.int32),
        ],
        compiler_params=_sc_params(),
    )
    def k(ps_hbm, zero_hbm, fs_hbm, fd_hbm, out_hbm, pt, acc, fs_v, fd_v):
        c = lax.axis_index("c")
        s = lax.axis_index("s")
        w = s * _NC + c
        one16i = jnp.ones((_L,), jnp.int32)

        pltpu.sync_copy(ps_hbm.at[w], pt)
        pltpu.sync_copy(zero_hbm, acc)

        def _chunk(ch, _):
            pltpu.sync_copy(fs_hbm.at[pl.ds(ch * _SCH, _SCH)], fs_v)
            pltpu.sync_copy(fd_hbm.at[pl.ds(ch * _SCH, _SCH)], fd_v)

            def _vec(v, __):
                gathered = []
                for u in range(_UNR):
                    o = v * _UNR * _L + u * _L
                    fs = fs_v[pl.ds(o, _L)]
                    fd = fd_v[pl.ds(o, _L)]
                    for _cc in range(_CS):
                        gathered.append((fd, plsc.load_gather(pt, [fs])))
                        if _cc + 1 < _CS:
                            fs = fs + one16i
                            fd = fd + one16i
                for fd, vals in gathered:
                    plsc.addupdate_scatter(acc, [fd], vals)
                return 0
            lax.fori_loop(0, NV // _UNR, _vec, 0)
            return 0
        lax.fori_loop(0, NCHK, _chunk, 0)

        pltpu.sync_copy(acc, out_hbm.at[w])

    return k


_RB = 1000


def _stage1_body(x_ref, pr_ref, w1l_ref, w1r_ref, b1_ref, h1_ref, cnt_ref):
    a = jnp.sum(pr_ref[:, :32], axis=1, keepdims=True)
    cnt_ref[...] = jnp.sum(pr_ref[:, 32:], axis=1, keepdims=True)
    h1 = a * w1l_ref[...] + x_ref[...] * w1r_ref[...] + b1_ref[...]
    h1_ref[...] = _leaky_relu(h1)


def _stage2_body(qh_ref, h_ref, b2_ref, Wl_ref, Wr_ref, h2_ref):
    dn = (((1,), (1,)), ((), ()))
    v = (lax.dot_general(qh_ref[...], Wl_ref[...], dn,
                         preferred_element_type=jnp.float32)
         + b2_ref[...]
         + lax.dot_general(h_ref[...], Wr_ref[...], dn,
                           preferred_element_type=jnp.float32))
    h2_ref[...] = _leaky_relu(v)


def _stage3_body(qh_ref, h_ref, cnt_ref, b3_ref, W3l_ref, W3r_ref, batch_ref,
                 wm_ref, wa_ref, blin_ref, out_ref, sums_ref, cntb_ref):
    i = pl.program_id(0)
    cnt = jnp.maximum(cnt_ref[...], 1.0)
    dn = (((1,), (1,)), ((), ()))
    h3 = (lax.dot_general(qh_ref[...] / cnt, W3l_ref[...], dn,
                          preferred_element_type=jnp.float32)
          + b3_ref[...]
          + lax.dot_general(h_ref[...], W3r_ref[...], dn,
                            preferred_element_type=jnp.float32))
    h3 = _leaky_relu(h3)
    b = batch_ref[0]
    gid = lax.broadcasted_iota(jnp.int32, (_G, h3.shape[0]), 0)
    onehot = (b == gid).astype(jnp.float32)
    ps = lax.dot_general(onehot, h3, (((1,), (0,)), ((), ())),
                         preferred_element_type=jnp.float32)
    pc = jnp.sum(onehot, axis=1, keepdims=True)

    @pl.when(i == 0)
    def _():
        sums_ref[...] = jnp.zeros_like(sums_ref)
        cntb_ref[...] = jnp.zeros_like(cntb_ref)

    sums_ref[...] += ps
    cntb_ref[:, 0:1] += pc

    @pl.when(i == pl.num_programs(0) - 1)
    def _():
        sums = sums_ref[...]
        cb = jnp.maximum(cntb_ref[:, 0:1], 1.0)
        z = (sums / cb) * wm_ref[...] + sums * wa_ref[...]
        out_ref[...] = jnp.sum(z, axis=1, keepdims=True) + blin_ref[...]


def _row_spec(w):
    return pl.BlockSpec((_RB, w), lambda i: (i, 0))


def _full_spec(shape):
    nd = len(shape)
    return pl.BlockSpec(shape, lambda i: (0,) * nd)


def _tc_params():
    return pltpu.CompilerParams(dimension_semantics=("arbitrary",))


@functools.lru_cache(maxsize=None)
def _stage1_call(N, H):
    grid = (N // _RB,)
    return pl.pallas_call(
        _stage1_body,
        grid=grid,
        in_specs=[_row_spec(1), _row_spec(64), _full_spec((1, H)),
                  _full_spec((1, H)), _full_spec((1, H))],
        out_specs=[_row_spec(H), _row_spec(1)],
        out_shape=[jax.ShapeDtypeStruct((N, H), jnp.float32),
                   jax.ShapeDtypeStruct((N, 1), jnp.float32)],
        compiler_params=_tc_params(),
    )


@functools.lru_cache(maxsize=None)
def _stage2_call(N, H):
    grid = (N // _RB,)
    return pl.pallas_call(
        _stage2_body,
        grid=grid,
        in_specs=[_row_spec(H), _row_spec(H), _full_spec((1, H)),
                  _full_spec((H, H)), _full_spec((H, H))],
        out_specs=[_row_spec(H)],
        out_shape=[jax.ShapeDtypeStruct((N, H), jnp.float32)],
        compiler_params=_tc_params(),
    )


@functools.lru_cache(maxsize=None)
def _stage3_call(N, H):
    grid = (N // _RB,)
    return pl.pallas_call(
        _stage3_body,
        grid=grid,
        in_specs=[_row_spec(H), _row_spec(H), _row_spec(1),
                  _full_spec((1, H)),
                  _full_spec((H, H)), _full_spec((H, H)),
                  pl.BlockSpec((1, 1, _RB), lambda i: (i, 0, 0)),
                  _full_spec((1, H)), _full_spec((1, H)),
                  _full_spec((1, 1))],
        out_specs=[_full_spec((_G, 1))],
        out_shape=[jax.ShapeDtypeStruct((_G, 1), jnp.float32)],
        scratch_shapes=[pltpu.VMEM((_G, H), jnp.float32),
                        pltpu.VMEM((_G, 128), jnp.float32)],
        compiler_params=_tc_params(),
    )


def _aggregate(p, eidx, N, E):
    H = p.shape[1]
    NS32 = _NC * _NS
    ps = p.reshape(N, H // _CS, _CS).transpose(1, 0, 2)
    ps = ps.reshape(H // _CS, N * _CS)
    ps = jnp.pad(ps, ((0, 0), (0, _ARB * 128 - N * _CS)))
    zeros = jnp.zeros((_ARB * 128,), jnp.float32)
    halves = []
    for h in range(H // (_CS * NS32)):
        qh = _sc_passB(N, E)(ps[h * NS32:(h + 1) * NS32], zeros, *eidx)
        qh = qh.reshape(NS32, _NP, _CS)[:, :N, :]
        halves.append(qh)
    q = jnp.concatenate(halves, axis=0)
    return q.transpose(1, 0, 2).reshape(N, H)


def kernel(x, edge_index, batch, W1l, b1, W1r, W2l, b2, W2r, W3l, b3, W3r,
           Wlin, blin):
    N = x.shape[0]
    E = edge_index.shape[1]
    H = W1l.shape[0]
    pad = _EP - E
    srcp = jnp.pad(edge_index[0], (0, pad))
    dstp = jnp.pad(edge_index[1], (0, pad),
                   constant_values=_NP - 8)

    xf = jnp.pad(x[:, 0], (0, _XR * 128 - N))
    zeros1 = jnp.zeros((_AR1 * 128,), jnp.float32)
    part = _sc_pass1(N, _EP)(xf, zeros1, srcp, dstp * 2)
    part = part[:, :_NP * 2].reshape(32, _NP, 2)
    pr = part[:, :N, :].transpose(1, 2, 0).reshape(N, 64)

    h1, cnt = _stage1_call(N, H)(x, pr, W1l.T, W1r.T, b1.reshape(1, H))
    eidx = (srcp * 4, dstp * 4)
    q1 = _aggregate(h1, eidx, N, _EP)

    h2, = _stage2_call(N, H)(q1, h1, b2.reshape(1, H), W2l, W2r)
    q2 = _aggregate(h2, eidx, N, _EP)

    out, = _stage3_call(N, H)(
        q2, h2, cnt, b3.reshape(1, H), W3l, W3r,
        batch.reshape(N // _RB, 1, _RB).astype(jnp.int32),
        Wlin[:, :H], Wlin[:, H:], blin.reshape(1, 1))
    return out
